# baseline (speedup 1.0000x reference)
import jax
import jax.numpy as jnp
from jax import lax
from jax.experimental import pallas as pl
from jax.experimental.pallas import tpu as pltpu

B = 64
EDD = 2048
EDS = 1024
CHUNK = 256
NCHUNK = EDS // CHUNK
BN_EPS = 1e-5
COS_EPS = 1e-8

_DN_T = (((1,), (1,)), ((), ()))
_DN = (((1,), (0,)), ((), ()))


def _fused_kernel(x_ref, y_ref, w_hbm, gx_ref, bx_ref, gy_ref, by_ref,
                  out_ref, wbuf, sems):
    copies = []
    for k in range(NCHUNK):
        c = pltpu.make_async_copy(
            w_hbm.at[pl.ds(k * CHUNK, CHUNK), :], wbuf.at[k], sems.at[k])
        c.start()
        copies.append(c)

    x = x_ref[...]
    y = y_ref[...]
    ones_row = jnp.ones((1, B), dtype=jnp.float32)
    ones_col = jnp.ones((CHUNK, 1), dtype=jnp.float32)
    lane = lax.broadcasted_iota(jnp.int32, (B, CHUNK), 1)
    at_block_start = (lane % 4) == 0
    neg_inf = jnp.full((B, CHUNK), -jnp.inf, dtype=jnp.float32)

    def bn_tanh(hh, g, bb):
        s1 = lax.dot_general(ones_row, hh, _DN,
                             preferred_element_type=jnp.float32)
        s2 = lax.dot_general(ones_row, hh * hh, _DN,
                             preferred_element_type=jnp.float32)
        mu = s1 * (1.0 / B)
        var = s2 * (1.0 / B) - mu * mu
        scale = lax.rsqrt(var + BN_EPS) * g
        shift = bb - mu * scale
        return jnp.tanh(hh * scale + shift)

    def block_mask(hh):
        a = jnp.maximum(hh, pltpu.roll(hh, CHUNK - 1, 1))
        bm = jnp.maximum(a, pltpu.roll(a, CHUNK - 2, 1))
        c = jnp.where(at_block_start, bm, neg_inf)
        c = jnp.maximum(c, pltpu.roll(c, 1, 1))
        bmax = jnp.maximum(c, pltpu.roll(c, 2, 1))
        return jnp.where(hh == bmax, hh, 0.0)

    dot = jnp.zeros((B, 1), dtype=jnp.float32)
    nx = jnp.zeros((B, 1), dtype=jnp.float32)
    ny = jnp.zeros((B, 1), dtype=jnp.float32)
    for k in range(NCHUNK):
        copies[k].wait()
        w = wbuf[k]
        cols = pl.ds(k * CHUNK, CHUNK)
        hx = lax.dot_general(x, w, _DN_T,
                             preferred_element_type=jnp.float32)
        hy = lax.dot_general(y, w, _DN_T,
                             preferred_element_type=jnp.float32)
        mx = block_mask(bn_tanh(hx, gx_ref[:, cols], bx_ref[:, cols]))
        my = block_mask(bn_tanh(hy, gy_ref[:, cols], by_ref[:, cols]))
        dot += lax.dot_general(mx * my, ones_col, _DN,
                               preferred_element_type=jnp.float32)
        nx += lax.dot_general(mx * mx, ones_col, _DN,
                              preferred_element_type=jnp.float32)
        ny += lax.dot_general(my * my, ones_col, _DN,
                              preferred_element_type=jnp.float32)

    nxc = jnp.maximum(jnp.sqrt(nx), COS_EPS)
    nyc = jnp.maximum(jnp.sqrt(ny), COS_EPS)
    out_ref[...] = dot / (nxc * nyc)


def kernel(x, y, W, b, gamma_x, beta_x, gamma_y, beta_y):
    row = lambda v: v.reshape(1, EDS)
    out = pl.pallas_call(
        _fused_kernel,
        in_specs=[
            pl.BlockSpec((B, EDD), lambda: (0, 0)),
            pl.BlockSpec((B, EDD), lambda: (0, 0)),
            pl.BlockSpec(memory_space=pltpu.MemorySpace.HBM),
            pl.BlockSpec((1, EDS), lambda: (0, 0)),
            pl.BlockSpec((1, EDS), lambda: (0, 0)),
            pl.BlockSpec((1, EDS), lambda: (0, 0)),
            pl.BlockSpec((1, EDS), lambda: (0, 0)),
        ],
        out_specs=pl.BlockSpec((B, 1), lambda: (0, 0)),
        out_shape=jax.ShapeDtypeStruct((B, 1), jnp.float32),
        scratch_shapes=[
            pltpu.VMEM((NCHUNK, CHUNK, EDD), jnp.float32),
            pltpu.SemaphoreType.DMA((NCHUNK,)),
        ],
    )(x, y, W, row(gamma_x), row(beta_x), row(gamma_y), row(beta_y))
    return out.reshape(B)
